# Initial kernel scaffold; baseline (speedup 1.0000x reference)
import jax
import jax.numpy as jnp
from jax.experimental import pallas as pl


def kernel(X, params):
    raise NotImplementedError("write your pallas kernel here")


# speedup vs baseline: 43.3692x; 43.3692x over previous
import functools

import jax
import jax.numpy as jnp
from jax import lax
from jax.experimental import pallas as pl
from jax.experimental.pallas import tpu as pltpu
from jax.experimental.pallas import tpu_sc as plsc

N_TOK = 2048
GROUPS = 512
ARITY = 4
OUT_DIM = 16
ENTRIES = 81
REF_IND = 40.0

NUM_CORES = 2
NUM_SUBCORES = 16
NUM_WORKERS = NUM_CORES * NUM_SUBCORES
GW = GROUPS // NUM_WORKERS
T_CHUNK = 256
N_CHUNKS = N_TOK // T_CHUNK
LANES = 16

_W_POS = (1.0, 3.0, 9.0, 27.0)
_NETWORK = ((0, 1), (2, 3), (0, 2), (1, 3), (1, 2))


def _sc_body(x_hbm, p_hbm, out_hbm, xbuf, pbuf, obuf):
    wid = lax.axis_index("s") * NUM_CORES + lax.axis_index("c")
    iota = lax.iota(jnp.int32, LANES)

    pltpu.sync_copy(p_hbm.at[pl.ds(wid * GW * ENTRIES, GW * ENTRIES)], pbuf)

    @pl.loop(0, N_CHUNKS)
    def _chunk(c):
        t_base = c * T_CHUNK
        pltpu.sync_copy(
            x_hbm.at[pl.ds(t_base, T_CHUNK), pl.ds(wid * (GW * ARITY), GW * ARITY)],
            xbuf,
        )

        @pl.loop(0, GW)
        def _group(gl):
            xcol = gl * ARITY
            prow = gl * ENTRIES
            ocol = gl * OUT_DIM

            @pl.loop(0, T_CHUNK, step=LANES)
            def _tile(t0):
                rows = t0 + iota
                sv = []
                wv = []
                for f in range(ARITY):
                    a = plsc.load_gather(
                        xbuf, [rows, jnp.full((LANES,), xcol + f, jnp.int32)]
                    )
                    sv.append(jnp.abs(a))
                    wv.append(jnp.where(a >= 0.0, _W_POS[f], -_W_POS[f]))
                for ia, ib in _NETWORK:
                    cmp = sv[ia] <= sv[ib]
                    lo_s = jnp.minimum(sv[ia], sv[ib])
                    hi_s = jnp.maximum(sv[ia], sv[ib])
                    lo_w = jnp.where(cmp, wv[ia], wv[ib])
                    hi_w = jnp.where(cmp, wv[ib], wv[ia])
                    sv[ia], sv[ib] = lo_s, hi_s
                    wv[ia], wv[ib] = lo_w, hi_w
                coef = (sv[0], sv[1] - sv[0], sv[2] - sv[1], sv[3] - sv[2])
                u3 = wv[3]
                u2 = u3 + wv[2]
                u1 = u2 + wv[1]
                u0 = u1 + wv[0]
                prows = [
                    prow + (REF_IND + u).astype(jnp.int32) for u in (u0, u1, u2, u3)
                ]
                for d in range(OUT_DIM):
                    dcol = jnp.full((LANES,), d, jnp.int32)
                    acc = coef[0] * plsc.load_gather(pbuf, [prows[0], dcol])
                    for j in range(1, ARITY):
                        acc += coef[j] * plsc.load_gather(pbuf, [prows[j], dcol])
                    plsc.store_scatter(
                        obuf, [rows, jnp.full((LANES,), ocol + d, jnp.int32)], acc
                    )

        pltpu.sync_copy(
            obuf,
            out_hbm.at[pl.ds(t_base, T_CHUNK), pl.ds(wid * (GW * OUT_DIM), GW * OUT_DIM)],
        )


@jax.jit
def _run(X, params_flat):
    mesh = plsc.VectorSubcoreMesh(
        core_axis_name="c",
        subcore_axis_name="s",
        num_cores=NUM_CORES,
        num_subcores=NUM_SUBCORES,
    )
    fn = pl.kernel(
        _sc_body,
        out_type=jax.ShapeDtypeStruct((N_TOK, GROUPS * OUT_DIM), jnp.float32),
        mesh=mesh,
        scratch_types=[
            pltpu.VMEM((T_CHUNK, GW * ARITY), jnp.float32),
            pltpu.VMEM((GW * ENTRIES, OUT_DIM), jnp.float32),
            pltpu.VMEM((T_CHUNK, GW * OUT_DIM), jnp.float32),
        ],
        compiler_params=pltpu.CompilerParams(
            use_tc_tiling_on_sc=False, needs_layout_passes=False
        ),
    )
    return fn(X, params_flat)


def kernel(X, params):
    return _run(X, params.reshape(GROUPS * ENTRIES, OUT_DIM))
